# baseline (speedup 1.0000x reference)
import functools

import jax
import jax.numpy as jnp
from jax.experimental import pallas as pl
from jax.experimental.pallas import tpu as pltpu

_RRELU_SLOPE = (1.0 / 8.0 + 1.0 / 3.0) / 2.0
_BN_EVAL_SCALE = 1.0 / (1.0 + 1e-5) ** 0.5

_F32 = jnp.float32
_BF16 = jnp.bfloat16


def _sigmoid(x):
    return 0.5 * jnp.tanh(0.5 * x) + 0.5


def _pre_kernel(ent_ref, rel_ref, wn_ref, wl0_ref,
                entw0_ref, loop0_ref, relw01_ref):
    wn0 = wn_ref[0]
    wn1 = wn_ref[1]
    ent = ent_ref[...]
    rel = rel_ref[...]
    entw0_ref[...] = jnp.dot(ent, wn0, preferred_element_type=_F32)
    loop0_ref[...] = jnp.dot(ent, wl0_ref[...], preferred_element_type=_F32)
    relw01_ref[...] = jnp.concatenate(
        [jnp.dot(rel, wn0, preferred_element_type=_F32),
         jnp.dot(rel, wn1, preferred_element_type=_F32)], axis=1)


def _row_gather(table_ref, idx_ref, out_ref, n):
    for e in range(n):
        i = idx_ref[0, 0, e]
        out_ref[pl.ds(e, 1), :] = table_ref[pl.ds(i, 1), :]


def _enc_kernel(src_ref, typ_ref, dst_ref, entw0_ref, loop0_ref, relw01_ref,
                wl1_ref, wn1_ref, wih_ref, whh_ref, brz_ref, bin_ref, bhn_ref,
                h2last_ref, statepre_ref,
                h2s_ref, state_ref, g0_ref, typm_ref, h1s_ref, g1_ref, *, T):
    t = pl.program_id(0)
    E = g0_ref.shape[0]
    Np, Dp = entw0_ref.shape

    x = h2s_ref[...]
    hprev = state_ref[...]
    hprevb = hprev.astype(_BF16)
    gi_r = jnp.dot(x, wih_ref[:, 0:Dp], preferred_element_type=_F32)
    gh_r = jnp.dot(hprevb, whh_ref[:, 0:Dp], preferred_element_type=_F32)
    gi_z = jnp.dot(x, wih_ref[:, Dp:2 * Dp], preferred_element_type=_F32)
    gh_z = jnp.dot(hprevb, whh_ref[:, Dp:2 * Dp], preferred_element_type=_F32)
    gi_n = jnp.dot(x, wih_ref[:, 2 * Dp:3 * Dp], preferred_element_type=_F32)
    gh_n = jnp.dot(hprevb, whh_ref[:, 2 * Dp:3 * Dp], preferred_element_type=_F32)
    r = _sigmoid(gi_r + gh_r + brz_ref[:, 0:Dp])
    z = _sigmoid(gi_z + gh_z + brz_ref[:, Dp:2 * Dp])
    n = jnp.tanh(gi_n + bin_ref[...] + r * (gh_n + bhn_ref[...]))
    state = n + z * (hprev - n)

    @pl.when(t == 0)
    def _():
        state_ref[...] = jnp.zeros_like(state_ref)

    @pl.when(t >= 1)
    def _():
        state_ref[...] = state

    @pl.when(t == T - 1)
    def _():
        statepre_ref[...] = state

    dst_row = dst_ref[0]
    dst_oh = (jax.lax.broadcasted_iota(jnp.int32, (Np, E), 0) == dst_row).astype(_BF16)
    deg = jnp.sum(dst_oh.astype(_F32), axis=1, keepdims=True)
    inv_deg = 1.0 / jnp.maximum(deg, 1.0)

    _row_gather(entw0_ref, src_ref, g0_ref, E)
    _row_gather(relw01_ref, typ_ref, typm_ref, E)

    typm = typm_ref[...]
    msg0 = g0_ref[...] + typm[:, 0:Dp]
    agg0 = jnp.dot(dst_oh, msg0.astype(_BF16), preferred_element_type=_F32) * inv_deg
    node0 = loop0_ref[...] + agg0
    h1 = jnp.maximum(node0, node0 * _RRELU_SLOPE)
    h1s_ref[...] = h1
    h1b = h1.astype(_BF16)

    loop1 = jnp.dot(h1b, wl1_ref[...], preferred_element_type=_F32)
    _row_gather(h1s_ref, src_ref, g1_ref, E)
    msg1 = (jnp.dot(g1_ref[...].astype(_BF16), wn1_ref[...],
                    preferred_element_type=_F32) + typm[:, Dp:2 * Dp])
    agg1 = jnp.dot(dst_oh, msg1.astype(_BF16), preferred_element_type=_F32) * inv_deg
    node1 = loop1 + agg1
    h2 = jnp.maximum(node1, node1 * _RRELU_SLOPE)
    h2b = h2.astype(_BF16)
    h2s_ref[...] = h2b

    @pl.when(t == T - 1)
    def _():
        h2last_ref[...] = h2b


def _dec_kernel(sub_ref, relidx_ref, obj_ref, h2_ref, state_ref,
                wih_ref, whh_ref, brz_ref, bin_ref, bhn_ref, relb_ref, fcw_ref,
                fcb_ref, convwb_ref, scores_ref, loss_ref, *, C):
    BB = sub_ref.shape[0]
    Np, Dp = h2_ref.shape
    Rp = relb_ref.shape[0]

    x = h2_ref[...]
    hprev = state_ref[...]
    hprevb = hprev.astype(_BF16)
    gi_r = jnp.dot(x, wih_ref[:, 0:Dp], preferred_element_type=_F32)
    gh_r = jnp.dot(hprevb, whh_ref[:, 0:Dp], preferred_element_type=_F32)
    gi_z = jnp.dot(x, wih_ref[:, Dp:2 * Dp], preferred_element_type=_F32)
    gh_z = jnp.dot(hprevb, whh_ref[:, Dp:2 * Dp], preferred_element_type=_F32)
    gi_n = jnp.dot(x, wih_ref[:, 2 * Dp:3 * Dp], preferred_element_type=_F32)
    gh_n = jnp.dot(hprevb, whh_ref[:, 2 * Dp:3 * Dp], preferred_element_type=_F32)
    r = _sigmoid(gi_r + gh_r + brz_ref[:, 0:Dp])
    z = _sigmoid(gi_z + gh_z + brz_ref[:, Dp:2 * Dp])
    n = jnp.tanh(gi_n + bin_ref[...] + r * (gh_n + bhn_ref[...]))
    state = n + z * (hprev - n)

    e1_all = jnp.tanh(state)
    e1b = e1_all.astype(_BF16)

    lane_n = jax.lax.broadcasted_iota(jnp.int32, (BB, Np), 1)
    lane_r = jax.lax.broadcasted_iota(jnp.int32, (BB, Rp), 1)
    sub_oh = (sub_ref[...] == lane_n).astype(_BF16)
    rel_oh = (relidx_ref[...] == lane_r).astype(_BF16)
    obj_oh = (obj_ref[...] == lane_n).astype(_F32)

    e1 = jnp.dot(sub_oh, e1b, preferred_element_type=_F32) * _BN_EVAL_SCALE
    rl = jnp.dot(rel_oh, relb_ref[...], preferred_element_type=_F32) * _BN_EVAL_SCALE

    lane_d = jax.lax.broadcasted_iota(jnp.int32, (BB, Dp), 1)
    zero = jnp.zeros((BB, Dp), _F32)

    def prev_tap(x):
        return jnp.where(lane_d >= 1, pltpu.roll(x, 1, 1), zero)

    def next_tap(x):
        return jnp.where(lane_d < Dp - 1, pltpu.roll(x, Dp - 1, 1), zero)

    taps = (prev_tap(e1), e1, next_tap(e1), prev_tap(rl), rl, next_tap(rl))

    conv_parts = []
    for c in range(C):
        acc = convwb_ref[c, 0] * taps[0]
        for j in range(1, 6):
            acc = acc + convwb_ref[c, j] * taps[j]
        acc = acc + convwb_ref[c, 6]
        conv_parts.append(jnp.maximum(acc * _BN_EVAL_SCALE, 0.0).astype(_BF16))
    conv_flat = jnp.concatenate(conv_parts, axis=1)

    fc = jnp.dot(conv_flat, fcw_ref[...], preferred_element_type=_F32) + fcb_ref[...]
    query = jnp.maximum(fc * _BN_EVAL_SCALE, 0.0).astype(_BF16)

    scores = jax.lax.dot_general(query, e1b, (((1,), (1,)), ((), ())),
                                 preferred_element_type=_F32)
    scores_ref[...] = scores

    m = jnp.max(scores, axis=-1, keepdims=True)
    lse = m + jnp.log(jnp.sum(jnp.exp(scores - m), axis=-1, keepdims=True))
    tgt = jnp.sum(scores * obj_oh, axis=-1, keepdims=True)
    loss_ref[...] = jnp.sum(lse - tgt, axis=0, keepdims=True)


def kernel(ent_embs, rel_embs, w_neighbor, w_loop, gru_w_ih, gru_w_hh,
           gru_b_ih, gru_b_hh, conv_w, conv_b, fc_w, fc_b,
           src_all, dst_all, etype_all, triplets):
    N, D = ent_embs.shape
    R = rel_embs.shape[0]
    T, E = src_all.shape
    B = triplets.shape[0]
    C = conv_w.shape[0]

    entb = ent_embs.astype(_BF16)
    relb = rel_embs.astype(_BF16)
    wnb = w_neighbor.astype(_BF16)
    wl0b = w_loop[0].astype(_BF16)
    wl1b = w_loop[1].astype(_BF16)
    wn1b = w_neighbor[1].astype(_BF16)

    entw0, loop0, relw01 = pl.pallas_call(
        _pre_kernel,
        out_shape=(jax.ShapeDtypeStruct((N, D), _F32),
                   jax.ShapeDtypeStruct((N, D), _F32),
                   jax.ShapeDtypeStruct((R, 2 * D), _F32)),
    )(entb, relb, wnb, wl0b)

    src_i = src_all.astype(jnp.int32)[:, None, :]
    typ_i = etype_all.astype(jnp.int32)[:, None, :]
    dst_arr = dst_all.astype(jnp.int32)[:, None, :]

    w3i = gru_w_ih.reshape(3, D, D)
    w3h = gru_w_hh.reshape(3, D, D)
    wih = jnp.transpose(w3i, (2, 0, 1)).reshape(D, 3 * D).astype(_BF16)
    whh = jnp.transpose(w3h, (2, 0, 1)).reshape(D, 3 * D).astype(_BF16)
    bih = gru_b_ih.reshape(1, 3 * D)
    bhh = gru_b_hh.reshape(1, 3 * D)
    brz = (bih + bhh)[:, 0:2 * D]
    bin_ = bih[:, 2 * D:3 * D]
    bhn = bhh[:, 2 * D:3 * D]

    smem = pltpu.MemorySpace.SMEM
    h2_last, state_pre = pl.pallas_call(
        functools.partial(_enc_kernel, T=T),
        out_shape=(jax.ShapeDtypeStruct((N, D), _BF16),
                   jax.ShapeDtypeStruct((N, D), _F32)),
        grid=(T,),
        in_specs=[
            pl.BlockSpec((1, 1, E), lambda t: (t, 0, 0), memory_space=smem),
            pl.BlockSpec((1, 1, E), lambda t: (t, 0, 0), memory_space=smem),
            pl.BlockSpec((1, 1, E), lambda t: (t, 0, 0)),
            pl.BlockSpec((N, D), lambda t: (0, 0)),
            pl.BlockSpec((N, D), lambda t: (0, 0)),
            pl.BlockSpec((R, 2 * D), lambda t: (0, 0)),
            pl.BlockSpec((D, D), lambda t: (0, 0)),
            pl.BlockSpec((D, D), lambda t: (0, 0)),
            pl.BlockSpec((D, 3 * D), lambda t: (0, 0)),
            pl.BlockSpec((D, 3 * D), lambda t: (0, 0)),
            pl.BlockSpec((1, 2 * D), lambda t: (0, 0)),
            pl.BlockSpec((1, D), lambda t: (0, 0)),
            pl.BlockSpec((1, D), lambda t: (0, 0)),
        ],
        out_specs=(pl.BlockSpec((N, D), lambda t: (0, 0)),
                   pl.BlockSpec((N, D), lambda t: (0, 0))),
        scratch_shapes=[pltpu.VMEM((N, D), _BF16),
                        pltpu.VMEM((N, D), _F32),
                        pltpu.VMEM((E, D), _F32),
                        pltpu.VMEM((E, 2 * D), _F32),
                        pltpu.VMEM((N, D), _F32),
                        pltpu.VMEM((E, D), _F32)],
        compiler_params=pltpu.CompilerParams(dimension_semantics=("arbitrary",)),
    )(src_i, typ_i, dst_arr, entw0, loop0, relw01,
      wl1b, wn1b, wih, whh, brz, bin_, bhn)

    sub = triplets[:, 0].astype(jnp.int32).reshape(B, 1)
    rel_idx = triplets[:, 1].astype(jnp.int32).reshape(B, 1)
    obj = triplets[:, 2].astype(jnp.int32).reshape(B, 1)

    fcw = fc_w.reshape(C * D, D).astype(_BF16)
    fcb = fc_b.reshape(1, D)
    conv_wb = jnp.concatenate([conv_w.reshape(C, 6), conv_b[:, None]], axis=1)

    vmem = pl.BlockSpec(memory_space=pltpu.MemorySpace.VMEM)
    smem_b = pl.BlockSpec(memory_space=pltpu.MemorySpace.SMEM)
    scores, loss_sum = pl.pallas_call(
        functools.partial(_dec_kernel, C=C),
        out_shape=(jax.ShapeDtypeStruct((B, N), _F32),
                   jax.ShapeDtypeStruct((1, 1), _F32)),
        in_specs=[vmem] * 13 + [smem_b],
        out_specs=(vmem, vmem),
    )(sub, rel_idx, obj, h2_last, state_pre, wih, whh, brz, bin_, bhn,
      relb, fcw, fcb, conv_wb)

    loss = loss_sum[0, 0] / B
    return loss, scores


# speedup vs baseline: 1.0767x; 1.0767x over previous
import functools

import jax
import jax.numpy as jnp
from jax.experimental import pallas as pl
from jax.experimental.pallas import tpu as pltpu

_RRELU_SLOPE = (1.0 / 8.0 + 1.0 / 3.0) / 2.0
_BN_EVAL_SCALE = 1.0 / (1.0 + 1e-5) ** 0.5

_F32 = jnp.float32
_BF16 = jnp.bfloat16


def _sigmoid(x):
    return 0.5 * jnp.tanh(0.5 * x) + 0.5


def _pre_kernel(ent_ref, rel_ref, wn_ref, wl0_ref,
                entw0_ref, loop0_ref, relw01_ref):
    wn0 = wn_ref[0].astype(_BF16)
    wn1 = wn_ref[1].astype(_BF16)
    ent = ent_ref[...].astype(_BF16)
    rel = rel_ref[...].astype(_BF16)
    wl0 = wl0_ref[...].astype(_BF16)
    entw0_ref[...] = jnp.dot(ent, wn0, preferred_element_type=_F32)
    loop0_ref[...] = jnp.dot(ent, wl0, preferred_element_type=_F32)
    relw01_ref[...] = jnp.concatenate(
        [jnp.dot(rel, wn0, preferred_element_type=_F32),
         jnp.dot(rel, wn1, preferred_element_type=_F32)], axis=1)


def _row_gather(table_ref, idx_ref, out_ref, n):
    for e in range(n):
        i = idx_ref[0, 0, e]
        out_ref[pl.ds(e, 1), :] = table_ref[pl.ds(i, 1), :]


def _enc_kernel(src_ref, typ_ref, dst_ref, entw0_ref, loop0_ref, relw01_ref,
                wl1_ref, wn1_ref, wih_ref, whh_ref, brz_ref, bin_ref, bhn_ref,
                h2last_ref, statepre_ref,
                h2s_ref, state_ref, g0_ref, typm_ref, h1s_ref, g1_ref, *, T):
    t = pl.program_id(0)
    E = g0_ref.shape[0]
    Np, Dp = entw0_ref.shape

    x = h2s_ref[...]
    hprev = state_ref[...]
    hprevb = hprev.astype(_BF16)
    gi_r = jnp.dot(x, wih_ref[:, 0:Dp], preferred_element_type=_F32)
    gh_r = jnp.dot(hprevb, whh_ref[:, 0:Dp], preferred_element_type=_F32)
    gi_z = jnp.dot(x, wih_ref[:, Dp:2 * Dp], preferred_element_type=_F32)
    gh_z = jnp.dot(hprevb, whh_ref[:, Dp:2 * Dp], preferred_element_type=_F32)
    gi_n = jnp.dot(x, wih_ref[:, 2 * Dp:3 * Dp], preferred_element_type=_F32)
    gh_n = jnp.dot(hprevb, whh_ref[:, 2 * Dp:3 * Dp], preferred_element_type=_F32)
    r = _sigmoid(gi_r + gh_r + brz_ref[:, 0:Dp])
    z = _sigmoid(gi_z + gh_z + brz_ref[:, Dp:2 * Dp])
    n = jnp.tanh(gi_n + bin_ref[...] + r * (gh_n + bhn_ref[...]))
    state = n + z * (hprev - n)

    @pl.when(t == 0)
    def _():
        state_ref[...] = jnp.zeros_like(state_ref)

    @pl.when(t >= 1)
    def _():
        state_ref[...] = state

    @pl.when(t == T - 1)
    def _():
        statepre_ref[...] = state

    dst_row = dst_ref[0]
    dst_oh = (jax.lax.broadcasted_iota(jnp.int32, (Np, E), 0) == dst_row).astype(_BF16)
    deg = jnp.sum(dst_oh.astype(_F32), axis=1, keepdims=True)
    inv_deg = 1.0 / jnp.maximum(deg, 1.0)

    _row_gather(entw0_ref, src_ref, g0_ref, E)
    _row_gather(relw01_ref, typ_ref, typm_ref, E)

    typm = typm_ref[...]
    msg0 = g0_ref[...] + typm[:, 0:Dp]
    agg0 = jnp.dot(dst_oh, msg0.astype(_BF16), preferred_element_type=_F32) * inv_deg
    node0 = loop0_ref[...] + agg0
    h1 = jnp.maximum(node0, node0 * _RRELU_SLOPE)
    h1s_ref[...] = h1
    h1b = h1.astype(_BF16)

    loop1 = jnp.dot(h1b, wl1_ref[...], preferred_element_type=_F32)
    _row_gather(h1s_ref, src_ref, g1_ref, E)
    msg1 = (jnp.dot(g1_ref[...].astype(_BF16), wn1_ref[...],
                    preferred_element_type=_F32) + typm[:, Dp:2 * Dp])
    agg1 = jnp.dot(dst_oh, msg1.astype(_BF16), preferred_element_type=_F32) * inv_deg
    node1 = loop1 + agg1
    h2 = jnp.maximum(node1, node1 * _RRELU_SLOPE)
    h2b = h2.astype(_BF16)
    h2s_ref[...] = h2b

    @pl.when(t == T - 1)
    def _():
        h2last_ref[...] = h2b


def _dec_kernel(sub_ref, relidx_ref, obj_ref, h2_ref, state_ref,
                wih_ref, whh_ref, brz_ref, bin_ref, bhn_ref, relb_ref, fcw_ref,
                fcb_ref, convwb_ref, scores_ref, loss_ref, *, C):
    BB = sub_ref.shape[0]
    Np, Dp = h2_ref.shape
    Rp = relb_ref.shape[0]

    x = h2_ref[...]
    hprev = state_ref[...]
    hprevb = hprev.astype(_BF16)
    gi_r = jnp.dot(x, wih_ref[:, 0:Dp], preferred_element_type=_F32)
    gh_r = jnp.dot(hprevb, whh_ref[:, 0:Dp], preferred_element_type=_F32)
    gi_z = jnp.dot(x, wih_ref[:, Dp:2 * Dp], preferred_element_type=_F32)
    gh_z = jnp.dot(hprevb, whh_ref[:, Dp:2 * Dp], preferred_element_type=_F32)
    gi_n = jnp.dot(x, wih_ref[:, 2 * Dp:3 * Dp], preferred_element_type=_F32)
    gh_n = jnp.dot(hprevb, whh_ref[:, 2 * Dp:3 * Dp], preferred_element_type=_F32)
    r = _sigmoid(gi_r + gh_r + brz_ref[:, 0:Dp])
    z = _sigmoid(gi_z + gh_z + brz_ref[:, Dp:2 * Dp])
    n = jnp.tanh(gi_n + bin_ref[...] + r * (gh_n + bhn_ref[...]))
    state = n + z * (hprev - n)

    e1_all = jnp.tanh(state)
    e1b = e1_all.astype(_BF16)

    lane_n = jax.lax.broadcasted_iota(jnp.int32, (BB, Np), 1)
    lane_r = jax.lax.broadcasted_iota(jnp.int32, (BB, Rp), 1)
    sub_oh = (sub_ref[...] == lane_n).astype(_BF16)
    rel_oh = (relidx_ref[...] == lane_r).astype(_BF16)
    obj_oh = (obj_ref[...] == lane_n).astype(_F32)

    e1 = jnp.dot(sub_oh, e1b, preferred_element_type=_F32) * _BN_EVAL_SCALE
    rl = (jnp.dot(rel_oh, relb_ref[...].astype(_BF16), preferred_element_type=_F32)
          * _BN_EVAL_SCALE)

    lane_d = jax.lax.broadcasted_iota(jnp.int32, (BB, Dp), 1)
    zero = jnp.zeros((BB, Dp), _F32)

    def prev_tap(x):
        return jnp.where(lane_d >= 1, pltpu.roll(x, 1, 1), zero)

    def next_tap(x):
        return jnp.where(lane_d < Dp - 1, pltpu.roll(x, Dp - 1, 1), zero)

    taps = (prev_tap(e1), e1, next_tap(e1), prev_tap(rl), rl, next_tap(rl))

    conv_parts = []
    for c in range(C):
        acc = convwb_ref[c, 0] * taps[0]
        for j in range(1, 6):
            acc = acc + convwb_ref[c, j] * taps[j]
        acc = acc + convwb_ref[c, 6]
        conv_parts.append(jnp.maximum(acc * _BN_EVAL_SCALE, 0.0).astype(_BF16))
    conv_flat = jnp.concatenate(conv_parts, axis=1)

    fc = (jnp.dot(conv_flat, fcw_ref[...].astype(_BF16),
                  preferred_element_type=_F32) + fcb_ref[...])
    query = jnp.maximum(fc * _BN_EVAL_SCALE, 0.0).astype(_BF16)

    scores = jax.lax.dot_general(query, e1b, (((1,), (1,)), ((), ())),
                                 preferred_element_type=_F32)
    scores_ref[...] = scores

    m = jnp.max(scores, axis=-1, keepdims=True)
    lse = m + jnp.log(jnp.sum(jnp.exp(scores - m), axis=-1, keepdims=True))
    tgt = jnp.sum(scores * obj_oh, axis=-1, keepdims=True)
    loss_ref[...] = jnp.sum(lse - tgt, axis=0, keepdims=True)


def kernel(ent_embs, rel_embs, w_neighbor, w_loop, gru_w_ih, gru_w_hh,
           gru_b_ih, gru_b_hh, conv_w, conv_b, fc_w, fc_b,
           src_all, dst_all, etype_all, triplets):
    N, D = ent_embs.shape
    R = rel_embs.shape[0]
    T, E = src_all.shape
    B = triplets.shape[0]
    C = conv_w.shape[0]

    wl1b = w_loop[1].astype(_BF16)
    wn1b = w_neighbor[1].astype(_BF16)

    entw0, loop0, relw01 = pl.pallas_call(
        _pre_kernel,
        out_shape=(jax.ShapeDtypeStruct((N, D), _F32),
                   jax.ShapeDtypeStruct((N, D), _F32),
                   jax.ShapeDtypeStruct((R, 2 * D), _F32)),
    )(ent_embs, rel_embs, w_neighbor, w_loop[0])

    src_i = src_all.astype(jnp.int32)[:, None, :]
    typ_i = etype_all.astype(jnp.int32)[:, None, :]
    dst_arr = dst_all.astype(jnp.int32)[:, None, :]

    w3i = gru_w_ih.reshape(3, D, D)
    w3h = gru_w_hh.reshape(3, D, D)
    wih = jnp.transpose(w3i, (2, 0, 1)).reshape(D, 3 * D).astype(_BF16)
    whh = jnp.transpose(w3h, (2, 0, 1)).reshape(D, 3 * D).astype(_BF16)
    bih = gru_b_ih.reshape(1, 3 * D)
    bhh = gru_b_hh.reshape(1, 3 * D)
    brz = (bih + bhh)[:, 0:2 * D]
    bin_ = bih[:, 2 * D:3 * D]
    bhn = bhh[:, 2 * D:3 * D]

    smem = pltpu.MemorySpace.SMEM
    h2_last, state_pre = pl.pallas_call(
        functools.partial(_enc_kernel, T=T),
        out_shape=(jax.ShapeDtypeStruct((N, D), _BF16),
                   jax.ShapeDtypeStruct((N, D), _F32)),
        grid=(T,),
        in_specs=[
            pl.BlockSpec((1, 1, E), lambda t: (t, 0, 0), memory_space=smem),
            pl.BlockSpec((1, 1, E), lambda t: (t, 0, 0), memory_space=smem),
            pl.BlockSpec((1, 1, E), lambda t: (t, 0, 0)),
            pl.BlockSpec((N, D), lambda t: (0, 0)),
            pl.BlockSpec((N, D), lambda t: (0, 0)),
            pl.BlockSpec((R, 2 * D), lambda t: (0, 0)),
            pl.BlockSpec((D, D), lambda t: (0, 0)),
            pl.BlockSpec((D, D), lambda t: (0, 0)),
            pl.BlockSpec((D, 3 * D), lambda t: (0, 0)),
            pl.BlockSpec((D, 3 * D), lambda t: (0, 0)),
            pl.BlockSpec((1, 2 * D), lambda t: (0, 0)),
            pl.BlockSpec((1, D), lambda t: (0, 0)),
            pl.BlockSpec((1, D), lambda t: (0, 0)),
        ],
        out_specs=(pl.BlockSpec((N, D), lambda t: (0, 0)),
                   pl.BlockSpec((N, D), lambda t: (0, 0))),
        scratch_shapes=[pltpu.VMEM((N, D), _BF16),
                        pltpu.VMEM((N, D), _F32),
                        pltpu.VMEM((E, D), _F32),
                        pltpu.VMEM((E, 2 * D), _F32),
                        pltpu.VMEM((N, D), _F32),
                        pltpu.VMEM((E, D), _F32)],
        compiler_params=pltpu.CompilerParams(dimension_semantics=("arbitrary",)),
    )(src_i, typ_i, dst_arr, entw0, loop0, relw01,
      wl1b, wn1b, wih, whh, brz, bin_, bhn)

    sub = triplets[:, 0].astype(jnp.int32).reshape(B, 1)
    rel_idx = triplets[:, 1].astype(jnp.int32).reshape(B, 1)
    obj = triplets[:, 2].astype(jnp.int32).reshape(B, 1)

    fcw = fc_w.reshape(C * D, D)
    fcb = fc_b.reshape(1, D)
    conv_wb = jnp.concatenate([conv_w.reshape(C, 6), conv_b[:, None]], axis=1)

    vmem = pl.BlockSpec(memory_space=pltpu.MemorySpace.VMEM)
    smem_b = pl.BlockSpec(memory_space=pltpu.MemorySpace.SMEM)
    scores, loss_sum = pl.pallas_call(
        functools.partial(_dec_kernel, C=C),
        out_shape=(jax.ShapeDtypeStruct((B, N), _F32),
                   jax.ShapeDtypeStruct((1, 1), _F32)),
        in_specs=[vmem] * 13 + [smem_b],
        out_specs=(vmem, vmem),
    )(sub, rel_idx, obj, h2_last, state_pre, wih, whh, brz, bin_, bhn,
      rel_embs, fcw, fcb, conv_wb)

    loss = loss_sum[0, 0] / B
    return loss, scores
